# baseline (speedup 1.0000x reference)
import dataclasses
import functools

import jax
import jax.numpy as jnp
from jax.experimental import pallas as pl
from jax.experimental.pallas import tpu as pltpu
from jax.experimental.pallas import tpu_sc as plsc

VOL = (256, 256, 256)
K = 256

ROWS = 65536
COLS = 256

SC_BLK_R = 64
SC_GRID = ROWS // SC_BLK_R
LANES = 16
NWORK = 32

_HIMASK = -65536


def _sc_mesh_and_params():
    mesh = plsc.VectorSubcoreMesh(core_axis_name="c", subcore_axis_name="s")
    cp = pltpu.CompilerParams(use_tc_tiling_on_sc=True)
    if "needs_layout_passes" in pltpu.CompilerParams.__dataclass_fields__:
        cp = dataclasses.replace(cp, needs_layout_passes=False)
    return mesh, cp


def _gather_intensity(tab_ref, lab_ref, noi_ref, r, c):
    idx = lab_ref[r, pl.ds(c, LANES)]
    w = plsc.load_gather(tab_ref, [idx])
    m = jax.lax.bitcast_convert_type(w & jnp.int32(_HIMASK), jnp.float32)
    s = jax.lax.bitcast_convert_type(w << 16, jnp.float32)
    return m + s * noi_ref[r, pl.ds(c, LANES)]


def _minmax_pipeline_body(tab_ref, acc_ref, lab_ref, noi_ref):
    inf = jnp.float32(jnp.inf)
    carry0 = (jnp.full((LANES,), inf, jnp.float32),
              jnp.full((LANES,), -inf, jnp.float32))

    @plsc.parallel_loop(0, SC_BLK_R, step=1, unroll=4, carry=carry0)
    def loop(r, carry):
        mn, mx = carry
        for c in range(0, COLS, LANES):
            x = _gather_intensity(tab_ref, lab_ref, noi_ref, r, c)
            mn = jnp.minimum(mn, x)
            mx = jnp.maximum(mx, jnp.where(x == inf, -inf, x))
        return mn, mx

    mn, mx = loop
    acc_ref[pl.ds(0, LANES)] = jnp.minimum(acc_ref[pl.ds(0, LANES)], mn)
    acc_ref[pl.ds(LANES, LANES)] = jnp.maximum(acc_ref[pl.ds(LANES, LANES)], mx)


def _sc_minmax(table, labels2d, noise2d):
    mesh, cp = _sc_mesh_and_params()

    @functools.partial(
        pl.kernel,
        out_type=jax.ShapeDtypeStruct((NWORK, 128), jnp.float32),
        mesh=mesh,
        compiler_params=cp,
        scratch_types=[
            pltpu.VMEM((K,), jnp.int32),
            pltpu.VMEM((128,), jnp.float32),
        ],
    )
    def sc_kernel(tab_hbm, lab_hbm, noi_hbm, part_hbm, tab_v, acc_v):
        pltpu.sync_copy(tab_hbm, tab_v)
        inf = jnp.float32(jnp.inf)
        for j in range(0, 128, LANES):
            acc_v[pl.ds(j, LANES)] = jnp.full((LANES,), inf, jnp.float32)
        acc_v[pl.ds(LANES, LANES)] = jnp.full((LANES,), -inf, jnp.float32)
        pltpu.emit_pipeline(
            functools.partial(_minmax_pipeline_body, tab_v, acc_v),
            grid=(SC_GRID,),
            in_specs=[
                pl.BlockSpec((SC_BLK_R, COLS), lambda i: (i, 0)),
                pl.BlockSpec((SC_BLK_R, COLS), lambda i: (i, 0)),
            ],
            out_specs=[],
            core_axis_name=("c", "s"),
            dimension_semantics=(pltpu.PARALLEL,),
        )(lab_hbm, noi_hbm)
        wid = jax.lax.axis_index("s") * 2 + jax.lax.axis_index("c")
        pltpu.sync_copy(acc_v, part_hbm.at[wid])

    return sc_kernel(table, labels2d, noise2d)


def _table_body(p_ref, me_ref, st_ref, tab_ref):
    p = p_ref[...]
    col = jax.lax.broadcasted_iota(jnp.int32, p.shape, 1)
    mn = jnp.min(jnp.where(col < LANES, p, jnp.inf))
    mx = jnp.max(jnp.where((col >= LANES) & (col < 2 * LANES), p, -jnp.inf))
    scale = 0.4 / (mx - mn)
    off = jnp.float32(0.1) - mn * scale

    me = me_ref[...]
    st = st_ref[...]
    mp = me * scale + off
    sp = st * scale
    row = jax.lax.broadcasted_iota(jnp.int32, me.shape, 0)
    colt = jax.lax.broadcasted_iota(jnp.int32, me.shape, 1)
    is0 = (row == 0) & (colt == 0)
    mp = jnp.where(is0, jnp.float32(0.0), mp)
    sp = jnp.where(is0, jnp.float32(0.0), sp)
    mb = jax.lax.bitcast_convert_type(
        mp.astype(jnp.bfloat16), jnp.uint16
    ).astype(jnp.uint32)
    sb = jax.lax.bitcast_convert_type(
        sp.astype(jnp.bfloat16), jnp.uint16
    ).astype(jnp.uint32)
    tab_ref[...] = jax.lax.bitcast_convert_type((mb << 16) | sb, jnp.int32)


def _tc_build_table(partials, means, stds):
    return pl.pallas_call(
        _table_body,
        out_shape=jax.ShapeDtypeStruct((2, 128), jnp.int32),
    )(partials, means.reshape(2, 128), stds.reshape(2, 128))


def _out_pipeline_body(tab_ref, lab_ref, noi_ref, out_ref):
    @plsc.parallel_loop(0, SC_BLK_R, step=1, unroll=4)
    def _(r):
        for c in range(0, COLS, LANES):
            out_ref[r, pl.ds(c, LANES)] = _gather_intensity(
                tab_ref, lab_ref, noi_ref, r, c
            )


def _sc_apply(table2, labels2d, noise2d):
    mesh, cp = _sc_mesh_and_params()

    @functools.partial(
        pl.kernel,
        out_type=jax.ShapeDtypeStruct((ROWS, COLS), jnp.float32),
        mesh=mesh,
        compiler_params=cp,
        scratch_types=[pltpu.VMEM((K,), jnp.int32)],
    )
    def sc_kernel(tab_hbm, lab_hbm, noi_hbm, out_hbm, tab_v):
        pltpu.sync_copy(tab_hbm, tab_v)
        pltpu.emit_pipeline(
            functools.partial(_out_pipeline_body, tab_v),
            grid=(SC_GRID,),
            in_specs=[
                pl.BlockSpec((SC_BLK_R, COLS), lambda i: (i, 0)),
                pl.BlockSpec((SC_BLK_R, COLS), lambda i: (i, 0)),
            ],
            out_specs=[pl.BlockSpec((SC_BLK_R, COLS), lambda i: (i, 0))],
            core_axis_name=("c", "s"),
            dimension_semantics=(pltpu.PARALLEL,),
        )(lab_hbm, noi_hbm, out_hbm)

    return sc_kernel(table2, labels2d, noise2d)


def _packed_table(means, stds):
    means_inf = means.at[0].set(jnp.inf)
    mh = jax.lax.bitcast_convert_type(
        means_inf.astype(jnp.bfloat16), jnp.uint16
    ).astype(jnp.uint32)
    sh = jax.lax.bitcast_convert_type(
        stds.astype(jnp.bfloat16), jnp.uint16
    ).astype(jnp.uint32)
    return jax.lax.bitcast_convert_type((mh << 16) | sh, jnp.int32)


def kernel(labels, means, stds, noise):
    table = _packed_table(means, stds)
    labels2d = labels.reshape(ROWS, COLS).astype(jnp.int32)
    noise2d = noise.reshape(ROWS, COLS)

    partials = _sc_minmax(table, labels2d, noise2d)
    table2 = _tc_build_table(partials, means, stds).reshape(K)
    out = _sc_apply(table2, labels2d, noise2d)
    return out.reshape(VOL)


# speedup vs baseline: 1.5709x; 1.5709x over previous
import dataclasses
import functools

import jax
import jax.numpy as jnp
from jax.experimental import pallas as pl
from jax.experimental.pallas import tpu as pltpu
from jax.experimental.pallas import tpu_sc as plsc

VOL = (256, 256, 256)
K = 256

ROWS = 65536
COLS = 256

SC_BLK_R = 64
SC_GRID = ROWS // SC_BLK_R
LANES = 16
NWORK = 32

_HIMASK = -65536


def _sc_mesh_and_params():
    mesh = plsc.VectorSubcoreMesh(core_axis_name="c", subcore_axis_name="s")
    cp = pltpu.CompilerParams(use_tc_tiling_on_sc=True)
    if "needs_layout_passes" in pltpu.CompilerParams.__dataclass_fields__:
        cp = dataclasses.replace(cp, needs_layout_passes=False)
    return mesh, cp


def _gather_intensity(tab_ref, lab_ref, noi_ref, r, c):
    idx = lab_ref[r, pl.ds(c, LANES)]
    w = plsc.load_gather(tab_ref, [idx])
    m = jax.lax.bitcast_convert_type(w & jnp.int32(_HIMASK), jnp.float32)
    s = jax.lax.bitcast_convert_type(w << 16, jnp.float32)
    return m + s * noi_ref[r, pl.ds(c, LANES)]


def _minmax_pipeline_body(tab_ref, acc_ref, lab_ref, noi_ref):
    inf = jnp.float32(jnp.inf)
    carry0 = (jnp.full((LANES,), inf, jnp.float32),
              jnp.full((LANES,), -inf, jnp.float32))

    @plsc.parallel_loop(0, SC_BLK_R * (COLS // LANES), step=1, unroll=8,
                        carry=carry0)
    def loop(i, carry):
        mn, mx = carry
        r = i >> 4
        c = (i & 15) * LANES
        x = _gather_intensity(tab_ref, lab_ref, noi_ref, r, c)
        mn = jnp.minimum(mn, x)
        mx = jnp.maximum(mx, jnp.where(x == inf, -inf, x))
        return mn, mx

    mn, mx = loop
    acc_ref[pl.ds(0, LANES)] = jnp.minimum(acc_ref[pl.ds(0, LANES)], mn)
    acc_ref[pl.ds(LANES, LANES)] = jnp.maximum(acc_ref[pl.ds(LANES, LANES)], mx)


def _sc_minmax(table, labels2d, noise2d):
    mesh, cp = _sc_mesh_and_params()

    @functools.partial(
        pl.kernel,
        out_type=jax.ShapeDtypeStruct((NWORK, 128), jnp.float32),
        mesh=mesh,
        compiler_params=cp,
        scratch_types=[
            pltpu.VMEM((K,), jnp.int32),
            pltpu.VMEM((128,), jnp.float32),
        ],
    )
    def sc_kernel(tab_hbm, lab_hbm, noi_hbm, part_hbm, tab_v, acc_v):
        pltpu.sync_copy(tab_hbm, tab_v)
        inf = jnp.float32(jnp.inf)
        for j in range(0, 128, LANES):
            acc_v[pl.ds(j, LANES)] = jnp.full((LANES,), inf, jnp.float32)
        acc_v[pl.ds(LANES, LANES)] = jnp.full((LANES,), -inf, jnp.float32)
        pltpu.emit_pipeline(
            functools.partial(_minmax_pipeline_body, tab_v, acc_v),
            grid=(SC_GRID,),
            in_specs=[
                pl.BlockSpec((SC_BLK_R, COLS), lambda i: (i, 0)),
                pl.BlockSpec((SC_BLK_R, COLS), lambda i: (i, 0)),
            ],
            out_specs=[],
            core_axis_name=("c", "s"),
            dimension_semantics=(pltpu.PARALLEL,),
        )(lab_hbm, noi_hbm)
        wid = jax.lax.axis_index("s") * 2 + jax.lax.axis_index("c")
        pltpu.sync_copy(acc_v, part_hbm.at[wid])

    return sc_kernel(table, labels2d, noise2d)


def _table_body(p_ref, me_ref, st_ref, tab_ref):
    p = p_ref[...]
    col = jax.lax.broadcasted_iota(jnp.int32, p.shape, 1)
    mn = jnp.min(jnp.where(col < LANES, p, jnp.inf))
    mx = jnp.max(jnp.where((col >= LANES) & (col < 2 * LANES), p, -jnp.inf))
    scale = 0.4 / (mx - mn)
    off = jnp.float32(0.1) - mn * scale

    me = me_ref[...]
    st = st_ref[...]
    mp = me * scale + off
    sp = st * scale
    row = jax.lax.broadcasted_iota(jnp.int32, me.shape, 0)
    colt = jax.lax.broadcasted_iota(jnp.int32, me.shape, 1)
    is0 = (row == 0) & (colt == 0)
    mp = jnp.where(is0, jnp.float32(0.0), mp)
    sp = jnp.where(is0, jnp.float32(0.0), sp)
    mb = jax.lax.bitcast_convert_type(
        mp.astype(jnp.bfloat16), jnp.uint16
    ).astype(jnp.uint32)
    sb = jax.lax.bitcast_convert_type(
        sp.astype(jnp.bfloat16), jnp.uint16
    ).astype(jnp.uint32)
    tab_ref[...] = jax.lax.bitcast_convert_type((mb << 16) | sb, jnp.int32)


def _tc_build_table(partials, means, stds):
    return pl.pallas_call(
        _table_body,
        out_shape=jax.ShapeDtypeStruct((2, 128), jnp.int32),
    )(partials, means.reshape(2, 128), stds.reshape(2, 128))


def _out_pipeline_body(tab_ref, lab_ref, noi_ref, out_ref):
    @plsc.parallel_loop(0, SC_BLK_R * (COLS // LANES), step=1, unroll=8)
    def _(i):
        r = i >> 4
        c = (i & 15) * LANES
        out_ref[r, pl.ds(c, LANES)] = _gather_intensity(
            tab_ref, lab_ref, noi_ref, r, c
        )


def _sc_apply(table2, labels2d, noise2d):
    mesh, cp = _sc_mesh_and_params()

    @functools.partial(
        pl.kernel,
        out_type=jax.ShapeDtypeStruct((ROWS, COLS), jnp.float32),
        mesh=mesh,
        compiler_params=cp,
        scratch_types=[pltpu.VMEM((K,), jnp.int32)],
    )
    def sc_kernel(tab_hbm, lab_hbm, noi_hbm, out_hbm, tab_v):
        pltpu.sync_copy(tab_hbm, tab_v)
        pltpu.emit_pipeline(
            functools.partial(_out_pipeline_body, tab_v),
            grid=(SC_GRID,),
            in_specs=[
                pl.BlockSpec((SC_BLK_R, COLS), lambda i: (i, 0)),
                pl.BlockSpec((SC_BLK_R, COLS), lambda i: (i, 0)),
            ],
            out_specs=[pl.BlockSpec((SC_BLK_R, COLS), lambda i: (i, 0))],
            core_axis_name=("c", "s"),
            dimension_semantics=(pltpu.PARALLEL,),
        )(lab_hbm, noi_hbm, out_hbm)

    return sc_kernel(table2, labels2d, noise2d)


def _packed_table(means, stds):
    means_inf = means.at[0].set(jnp.inf)
    mh = jax.lax.bitcast_convert_type(
        means_inf.astype(jnp.bfloat16), jnp.uint16
    ).astype(jnp.uint32)
    sh = jax.lax.bitcast_convert_type(
        stds.astype(jnp.bfloat16), jnp.uint16
    ).astype(jnp.uint32)
    return jax.lax.bitcast_convert_type((mh << 16) | sh, jnp.int32)


def kernel(labels, means, stds, noise):
    table = _packed_table(means, stds)
    labels2d = labels.reshape(ROWS, COLS).astype(jnp.int32)
    noise2d = noise.reshape(ROWS, COLS)

    partials = _sc_minmax(table, labels2d, noise2d)
    table2 = _tc_build_table(partials, means, stds).reshape(K)
    out = _sc_apply(table2, labels2d, noise2d)
    return out.reshape(VOL)


# speedup vs baseline: 1.5757x; 1.0031x over previous
import dataclasses
import functools

import jax
import jax.numpy as jnp
from jax.experimental import pallas as pl
from jax.experimental.pallas import tpu as pltpu
from jax.experimental.pallas import tpu_sc as plsc

VOL = (256, 256, 256)
K = 256

ROWS = 65536
COLS = 256

SC_BLK_R = 64
SC_GRID = ROWS // SC_BLK_R
LANES = 16
NWORK = 32

_HIMASK = -65536


def _sc_mesh_and_params():
    mesh = plsc.VectorSubcoreMesh(core_axis_name="c", subcore_axis_name="s")
    cp = pltpu.CompilerParams(use_tc_tiling_on_sc=True)
    if "needs_layout_passes" in pltpu.CompilerParams.__dataclass_fields__:
        cp = dataclasses.replace(cp, needs_layout_passes=False)
    return mesh, cp


def _gather_intensity(tab_ref, lab_ref, noi_ref, r, c):
    idx = lab_ref[r, pl.ds(c, LANES)]
    w = plsc.load_gather(tab_ref, [idx])
    m = jax.lax.bitcast_convert_type(w & jnp.int32(_HIMASK), jnp.float32)
    s = jax.lax.bitcast_convert_type(w << 16, jnp.float32)
    return m + s * noi_ref[r, pl.ds(c, LANES)]


def _minmax_pipeline_body(tab_ref, acc_ref, lab_ref, noi_ref):
    inf = jnp.float32(jnp.inf)
    carry0 = (jnp.full((LANES,), inf, jnp.float32),
              jnp.full((LANES,), -inf, jnp.float32))

    @plsc.parallel_loop(0, SC_BLK_R * (COLS // LANES), step=1, unroll=16,
                        carry=carry0)
    def loop(i, carry):
        mn, mx = carry
        r = i >> 4
        c = (i & 15) * LANES
        x = _gather_intensity(tab_ref, lab_ref, noi_ref, r, c)
        mn = jnp.minimum(mn, x)
        mx = jnp.maximum(mx, jnp.where(x == inf, -inf, x))
        return mn, mx

    mn, mx = loop
    acc_ref[pl.ds(0, LANES)] = jnp.minimum(acc_ref[pl.ds(0, LANES)], mn)
    acc_ref[pl.ds(LANES, LANES)] = jnp.maximum(acc_ref[pl.ds(LANES, LANES)], mx)


def _sc_minmax(table, labels2d, noise2d):
    mesh, cp = _sc_mesh_and_params()

    @functools.partial(
        pl.kernel,
        out_type=jax.ShapeDtypeStruct((NWORK, 128), jnp.float32),
        mesh=mesh,
        compiler_params=cp,
        scratch_types=[
            pltpu.VMEM((K,), jnp.int32),
            pltpu.VMEM((128,), jnp.float32),
        ],
    )
    def sc_kernel(tab_hbm, lab_hbm, noi_hbm, part_hbm, tab_v, acc_v):
        pltpu.sync_copy(tab_hbm, tab_v)
        inf = jnp.float32(jnp.inf)
        for j in range(0, 128, LANES):
            acc_v[pl.ds(j, LANES)] = jnp.full((LANES,), inf, jnp.float32)
        acc_v[pl.ds(LANES, LANES)] = jnp.full((LANES,), -inf, jnp.float32)
        pltpu.emit_pipeline(
            functools.partial(_minmax_pipeline_body, tab_v, acc_v),
            grid=(SC_GRID,),
            in_specs=[
                pl.BlockSpec((SC_BLK_R, COLS), lambda i: (i, 0)),
                pl.BlockSpec((SC_BLK_R, COLS), lambda i: (i, 0)),
            ],
            out_specs=[],
            core_axis_name=("c", "s"),
            dimension_semantics=(pltpu.PARALLEL,),
        )(lab_hbm, noi_hbm)
        wid = jax.lax.axis_index("s") * 2 + jax.lax.axis_index("c")
        pltpu.sync_copy(acc_v, part_hbm.at[wid])

    return sc_kernel(table, labels2d, noise2d)


def _table_body(p_ref, me_ref, st_ref, tab_ref):
    p = p_ref[...]
    col = jax.lax.broadcasted_iota(jnp.int32, p.shape, 1)
    mn = jnp.min(jnp.where(col < LANES, p, jnp.inf))
    mx = jnp.max(jnp.where((col >= LANES) & (col < 2 * LANES), p, -jnp.inf))
    scale = 0.4 / (mx - mn)
    off = jnp.float32(0.1) - mn * scale

    me = me_ref[...]
    st = st_ref[...]
    mp = me * scale + off
    sp = st * scale
    row = jax.lax.broadcasted_iota(jnp.int32, me.shape, 0)
    colt = jax.lax.broadcasted_iota(jnp.int32, me.shape, 1)
    is0 = (row == 0) & (colt == 0)
    mp = jnp.where(is0, jnp.float32(0.0), mp)
    sp = jnp.where(is0, jnp.float32(0.0), sp)
    mb = jax.lax.bitcast_convert_type(
        mp.astype(jnp.bfloat16), jnp.uint16
    ).astype(jnp.uint32)
    sb = jax.lax.bitcast_convert_type(
        sp.astype(jnp.bfloat16), jnp.uint16
    ).astype(jnp.uint32)
    tab_ref[...] = jax.lax.bitcast_convert_type((mb << 16) | sb, jnp.int32)


def _tc_build_table(partials, means, stds):
    return pl.pallas_call(
        _table_body,
        out_shape=jax.ShapeDtypeStruct((2, 128), jnp.int32),
    )(partials, means.reshape(2, 128), stds.reshape(2, 128))


def _out_pipeline_body(tab_ref, lab_ref, noi_ref, out_ref):
    @plsc.parallel_loop(0, SC_BLK_R * (COLS // LANES), step=1, unroll=16)
    def _(i):
        r = i >> 4
        c = (i & 15) * LANES
        out_ref[r, pl.ds(c, LANES)] = _gather_intensity(
            tab_ref, lab_ref, noi_ref, r, c
        )


def _sc_apply(table2, labels2d, noise2d):
    mesh, cp = _sc_mesh_and_params()

    @functools.partial(
        pl.kernel,
        out_type=jax.ShapeDtypeStruct((ROWS, COLS), jnp.float32),
        mesh=mesh,
        compiler_params=cp,
        scratch_types=[pltpu.VMEM((K,), jnp.int32)],
    )
    def sc_kernel(tab_hbm, lab_hbm, noi_hbm, out_hbm, tab_v):
        pltpu.sync_copy(tab_hbm, tab_v)
        pltpu.emit_pipeline(
            functools.partial(_out_pipeline_body, tab_v),
            grid=(SC_GRID,),
            in_specs=[
                pl.BlockSpec((SC_BLK_R, COLS), lambda i: (i, 0)),
                pl.BlockSpec((SC_BLK_R, COLS), lambda i: (i, 0)),
            ],
            out_specs=[pl.BlockSpec((SC_BLK_R, COLS), lambda i: (i, 0))],
            core_axis_name=("c", "s"),
            dimension_semantics=(pltpu.PARALLEL,),
        )(lab_hbm, noi_hbm, out_hbm)

    return sc_kernel(table2, labels2d, noise2d)


def _packed_table(means, stds):
    means_inf = means.at[0].set(jnp.inf)
    mh = jax.lax.bitcast_convert_type(
        means_inf.astype(jnp.bfloat16), jnp.uint16
    ).astype(jnp.uint32)
    sh = jax.lax.bitcast_convert_type(
        stds.astype(jnp.bfloat16), jnp.uint16
    ).astype(jnp.uint32)
    return jax.lax.bitcast_convert_type((mh << 16) | sh, jnp.int32)


def kernel(labels, means, stds, noise):
    table = _packed_table(means, stds)
    labels2d = labels.reshape(ROWS, COLS).astype(jnp.int32)
    noise2d = noise.reshape(ROWS, COLS)

    partials = _sc_minmax(table, labels2d, noise2d)
    table2 = _tc_build_table(partials, means, stds).reshape(K)
    out = _sc_apply(table2, labels2d, noise2d)
    return out.reshape(VOL)
